# baseline (speedup 1.0000x reference)
import functools

import jax
import jax.numpy as jnp
from jax import lax
from jax.experimental import pallas as pl
from jax.experimental.pallas import tpu as pltpu
from jax.experimental.pallas import tpu_sc as plsc

_B = 8
_N = 2048
_S = 512
_RADII = (0.2, 0.4, 0.8)
_KS = (16, 32, 64)
_EPS = 1e-5


def _fps_body(xt_ref, nxo_ref):
    x = xt_ref[0]
    y = xt_ref[1]
    z = xt_ref[2]
    iota = lax.broadcasted_iota(jnp.int32, (_B, _N), 1)

    def step(i, carry):
        dist, far = carry
        oh = (iota == far).astype(jnp.float32)
        cx = jnp.sum(x * oh, axis=1, keepdims=True)
        cy = jnp.sum(y * oh, axis=1, keepdims=True)
        cz = jnp.sum(z * oh, axis=1, keepdims=True)
        nxo_ref[pl.ds(i, 1)] = jnp.concatenate([cx, cy, cz], axis=1)[None]
        dx = x - cx
        dy = y - cy
        dz = z - cz
        d = dx * dx + dy * dy + dz * dz
        dist = jnp.minimum(dist, d)
        m = jnp.max(dist, axis=1, keepdims=True)
        far = jnp.min(jnp.where(dist == m, iota, _N), axis=1, keepdims=True)
        return dist, far

    dist0 = jnp.full((_B, _N), jnp.inf, dtype=jnp.float32)
    far0 = jnp.zeros((_B, 1), dtype=jnp.int32)
    lax.fori_loop(0, _S, step, (dist0, far0))


def _fps(xyzt):
    return pl.pallas_call(
        _fps_body,
        out_shape=jax.ShapeDtypeStruct((_S, _B, 3), jnp.float32),
    )(xyzt)


def _bq_body(xt_ref, qt_ref, o1_ref, o2_ref, o3_ref):
    b = pl.program_id(0)
    xt = xt_ref[0]
    qt = qt_ref[0]
    xx = jnp.sum(xt * xt, axis=0)
    qq = jnp.sum(qt * qt, axis=0)
    prod = lax.dot_general(xt, qt, (((0,), (0,)), ((), ())),
                           preferred_element_type=jnp.float32)
    d = -2.0 * prod + qq[None, :] + xx[:, None]
    sq = jnp.maximum(d, 0.0)

    li = lax.broadcasted_iota(jnp.int32, (128, 128), 0)
    lj = lax.broadcasted_iota(jnp.int32, (128, 128), 1)
    lt128b = jnp.broadcast_to(((li <= lj).astype(jnp.float32))[None],
                              (16, 128, 128))
    si = lax.broadcasted_iota(jnp.int32, (16, 16), 0)
    sj = lax.broadcasted_iota(jnp.int32, (16, 16), 1)
    lts16 = (si < sj).astype(jnp.float32)

    for o_ref, r, k in ((o1_ref, _RADII[0], _KS[0]),
                        (o2_ref, _RADII[1], _KS[1]),
                        (o3_ref, _RADII[2], _KS[2])):
        mask = (sq <= r * r).astype(jnp.float32)
        m3 = mask.reshape(16, 128, _S)
        within = lax.dot_general(lt128b, m3, (((1,), (1,)), ((0,), (0,))),
                                 preferred_element_type=jnp.float32)
        tot = within[:, 127, :]
        carry = lax.dot_general(lts16, tot, (((0,), (0,)), ((), ())),
                                preferred_element_type=jnp.float32)
        c = (within + carry[:, None, :]).reshape(_N, _S)
        c16 = c.astype(jnp.int16)
        total = jnp.sum(tot, axis=0)

        def jstep(j, _, o_ref=o_ref, c16=c16):
            cnt = jnp.sum((c16 <= j.astype(jnp.int16)).astype(jnp.int16),
                          axis=0)
            o_ref[0, pl.ds(j, 1)] = cnt.astype(jnp.int32).reshape(1, _S)
            return 0

        lax.fori_loop(0, k, jstep, 0)
        nj = o_ref[0]
        jrow = lax.broadcasted_iota(jnp.int32, (k, _S), 0)
        trow = total.astype(jnp.int32).reshape(1, _S)
        o_ref[0] = jnp.where(jrow < trow, nj, nj[0:1]) + b * _N


def _ball_query(xt, qt):
    outs = [jax.ShapeDtypeStruct((_B, k, _S), jnp.int32) for k in _KS]
    return pl.pallas_call(
        _bq_body,
        grid=(_B,),
        in_specs=[
            pl.BlockSpec((1, 3, _N), lambda b: (b, 0, 0)),
            pl.BlockSpec((1, 3, _S), lambda b: (b, 0, 0)),
        ],
        out_specs=[pl.BlockSpec((1, k, _S), lambda b: (b, 0, 0)) for k in _KS],
        out_shape=outs,
    )(xt, qt)


_C1S = (32, 64, 64)
_SPLITS = (0, 32, 96, 160)


def _proj_body(xyz_ref, feat_ref, nx_ref, w1x_ref, w1f_ref,
               p1_ref, p2_ref, p3_ref, q1_ref, q2_ref, q3_ref):
    w1x = w1x_ref[...]
    w1f = w1f_ref[...]
    p = (lax.dot_general(xyz_ref[0], w1x, (((1,), (0,)), ((), ())),
                         preferred_element_type=jnp.float32)
         + lax.dot_general(feat_ref[0], w1f, (((1,), (0,)), ((), ())),
                           preferred_element_type=jnp.float32))
    q = lax.dot_general(nx_ref[0], w1x, (((1,), (0,)), ((), ())),
                        preferred_element_type=jnp.float32)
    for o_ref, lo, hi in ((p1_ref, 0, 32), (p2_ref, 32, 96), (p3_ref, 96, 160)):
        pad = jnp.zeros((_N, 128 - (hi - lo)), dtype=jnp.float32)
        o_ref[0] = jnp.concatenate([p[:, lo:hi], pad], axis=1)
    for o_ref, lo, hi in ((q1_ref, 0, 32), (q2_ref, 32, 96), (q3_ref, 96, 160)):
        o_ref[0] = q[:, lo:hi]


def _project(xyz, feat, nxyz, w1x, w1f):
    outs = ([jax.ShapeDtypeStruct((_B, _N, 128), jnp.float32) for c in _C1S]
            + [jax.ShapeDtypeStruct((_B, _S, c), jnp.float32) for c in _C1S])
    return pl.pallas_call(
        _proj_body,
        grid=(_B,),
        in_specs=[
            pl.BlockSpec((1, _N, 3), lambda b: (b, 0, 0)),
            pl.BlockSpec((1, _N, 64), lambda b: (b, 0, 0)),
            pl.BlockSpec((1, _S, 3), lambda b: (b, 0, 0)),
            pl.BlockSpec((3, 160), lambda b: (0, 0)),
            pl.BlockSpec((64, 160), lambda b: (0, 0)),
        ],
        out_specs=(
            [pl.BlockSpec((1, _N, 128), lambda b: (b, 0, 0)) for c in _C1S]
            + [pl.BlockSpec((1, _S, c), lambda b: (b, 0, 0)) for c in _C1S]),
        out_shape=outs,
    )(xyz, feat, nxyz, w1x, w1f)


def _sc_gather(table, idx, rows_total, c, chunk):
    nw = 32
    per_w = rows_total // nw
    n_chunks = per_w // chunk
    mesh = plsc.VectorSubcoreMesh(core_axis_name="c", subcore_axis_name="s")

    @functools.partial(
        pl.kernel, mesh=mesh,
        out_type=jax.ShapeDtypeStruct((rows_total, c), jnp.float32),
        scratch_types=[
            pltpu.VMEM((chunk,), jnp.int32),
            pltpu.VMEM((chunk, c), jnp.float32),
            pltpu.SemaphoreType.DMA,
        ],
    )
    def gk(table_hbm, idx_hbm, out_hbm, idx_v, rows_v, sem):
        wid = lax.axis_index("s") * 2 + lax.axis_index("c")
        base0 = wid * per_w

        def step(t, _):
            base = base0 + t * chunk
            pltpu.sync_copy(idx_hbm.at[pl.ds(base, chunk)], idx_v)
            pltpu.async_copy(table_hbm.at[idx_v], rows_v, sem).wait()
            pltpu.sync_copy(rows_v, out_hbm.at[pl.ds(base, chunk)])
            return 0

        lax.fori_loop(0, n_chunks, step, 0)

    return gk(table, idx)


def _bn_relu(y, st, gb, n_elems):
    mean = st[0:1] / n_elems
    var = st[1:2] / n_elems - mean * mean
    inv = lax.rsqrt(var + _EPS)
    return jnp.maximum((y - mean) * (inv * gb[0:1]) + gb[1:2], 0.0)


def _qtile(q_ref, kc):
    q = q_ref[0]
    return jnp.broadcast_to(q[None], (kc, _S, q.shape[-1])).reshape(
        kc * _S, q.shape[-1])


def _colstats(y):
    s = jnp.sum(y, axis=0, keepdims=True)
    s2 = jnp.sum(y * y, axis=0, keepdims=True)
    return jnp.concatenate([s, s2], axis=0)


def _acc_stats(o_ref, st):
    first = (pl.program_id(0) == 0) & (pl.program_id(1) == 0)

    @pl.when(first)
    def _():
        o_ref[...] = st

    @pl.when(jnp.logical_not(first))
    def _():
        o_ref[...] = o_ref[...] + st


def _forward(refs, kc, n_elems, depth):
    g_ref, q_ref = refs[0], refs[1]
    c1 = q_ref.shape[-1]
    h = g_ref[...][:, :c1] - _qtile(q_ref, kc)
    i = 2
    for layer in range(depth):
        st = refs[i][...]
        gb = refs[i + 1][...]
        h = _bn_relu(h, st, gb, n_elems)
        i += 2
        if layer < depth - 1:
            wt = refs[i][...]
            i += 1
            h = lax.dot_general(h, wt, (((1,), (0,)), ((), ())),
                                preferred_element_type=jnp.float32)
    return h


def _make_e1(kc):
    def body(g_ref, q_ref, o_ref):
        c1 = q_ref.shape[-1]
        y = g_ref[...][:, :c1] - _qtile(q_ref, kc)
        _acc_stats(o_ref, _colstats(y))
    return body


def _make_e2(kc, n_elems):
    def body(g_ref, q_ref, st1_ref, gb1_ref, w2t_ref, o_ref):
        h = _forward((g_ref, q_ref, st1_ref, gb1_ref), kc, n_elems, 1)
        y2 = lax.dot_general(h, w2t_ref[...], (((1,), (0,)), ((), ())),
                             preferred_element_type=jnp.float32)
        _acc_stats(o_ref, _colstats(y2))
    return body


def _make_e3(kc, n_elems):
    def body(g_ref, q_ref, st1_ref, gb1_ref, w2t_ref, st2_ref, gb2_ref,
             w3t_ref, o_ref):
        h = _forward((g_ref, q_ref, st1_ref, gb1_ref, w2t_ref, st2_ref,
                      gb2_ref), kc, n_elems, 2)
        y3 = lax.dot_general(h, w3t_ref[...], (((1,), (0,)), ((), ())),
                             preferred_element_type=jnp.float32)
        _acc_stats(o_ref, _colstats(y3))
    return body


def _make_e4(kc, n_elems):
    def body(g_ref, q_ref, st1_ref, gb1_ref, w2t_ref, st2_ref, gb2_ref,
             w3t_ref, st3_ref, gb3_ref, o_ref):
        h = _forward((g_ref, q_ref, st1_ref, gb1_ref, w2t_ref, st2_ref,
                      gb2_ref, w3t_ref, st3_ref, gb3_ref), kc, n_elems, 3)
        part = jnp.max(h.reshape(kc, _S, h.shape[-1]), axis=0)[None]

        @pl.when(pl.program_id(1) == 0)
        def _():
            o_ref[...] = part

        @pl.when(pl.program_id(1) != 0)
        def _():
            o_ref[...] = jnp.maximum(o_ref[...], part)
    return body


def _mlp_scale(g, q, layer_params, k, kc):
    n_elems = float(_B * k * _S)
    (w1, g1, b1), (w2, g2, b2), (w3, g3, b3) = layer_params
    c1, c2, c3 = w1.shape[0], w2.shape[0], w3.shape[0]
    gb1 = jnp.stack([g1, b1])
    gb2 = jnp.stack([g2, b2])
    gb3 = jnp.stack([g3, b3])
    w2t = jnp.transpose(w2)
    w3t = jnp.transpose(w3)
    grid = (_B, k // kc)
    rows = kc * _S

    g_spec = pl.BlockSpec((rows, 128),
                          lambda b, t, nt=k // kc: (b * nt + t, 0))
    q_spec = pl.BlockSpec((1, _S, c1), lambda b, t: (b, 0, 0))

    def full(shape):
        nd = len(shape)
        return pl.BlockSpec(shape, lambda b, t, nd=nd: (0,) * nd)

    st1 = pl.pallas_call(
        _make_e1(kc), grid=grid,
        in_specs=[g_spec, q_spec],
        out_specs=pl.BlockSpec((2, c1), lambda b, t: (0, 0)),
        out_shape=jax.ShapeDtypeStruct((2, c1), jnp.float32),
    )(g, q)

    st2 = pl.pallas_call(
        _make_e2(kc, n_elems), grid=grid,
        in_specs=[g_spec, q_spec, full((2, c1)), full((2, c1)),
                  full((c1, c2))],
        out_specs=pl.BlockSpec((2, c2), lambda b, t: (0, 0)),
        out_shape=jax.ShapeDtypeStruct((2, c2), jnp.float32),
    )(g, q, st1, gb1, w2t)

    st3 = pl.pallas_call(
        _make_e3(kc, n_elems), grid=grid,
        in_specs=[g_spec, q_spec, full((2, c1)), full((2, c1)),
                  full((c1, c2)), full((2, c2)), full((2, c2)),
                  full((c2, c3))],
        out_specs=pl.BlockSpec((2, c3), lambda b, t: (0, 0)),
        out_shape=jax.ShapeDtypeStruct((2, c3), jnp.float32),
    )(g, q, st1, gb1, w2t, st2, gb2, w3t)

    out = pl.pallas_call(
        _make_e4(kc, n_elems), grid=grid,
        in_specs=[g_spec, q_spec, full((2, c1)), full((2, c1)),
                  full((c1, c2)), full((2, c2)), full((2, c2)),
                  full((c2, c3)), full((2, c3)), full((2, c3))],
        out_specs=pl.BlockSpec((1, _S, c3), lambda b, t: (b, 0, 0)),
        out_shape=jax.ShapeDtypeStruct((_B, _S, c3), jnp.float32),
    )(g, q, st1, gb1, w2t, st2, gb2, w3t, st3, gb3)
    return out


def kernel(xyz, features, params):
    xyzt = jnp.transpose(xyz, (2, 0, 1))
    nx_sb3 = _fps(xyzt)
    new_xyz = jnp.transpose(nx_sb3, (1, 0, 2))

    xt = jnp.transpose(xyz, (0, 2, 1))
    qt = jnp.transpose(new_xyz, (0, 2, 1))
    idxs_ks = _ball_query(xt, qt)
    idxs = [ix.reshape(-1) for ix in idxs_ks]

    w1cat = jnp.concatenate([lp[0][0] for lp in params], axis=0)
    w1x = jnp.transpose(w1cat[:, :3])
    w1f = jnp.transpose(w1cat[:, 3:])
    p1, p2, p3, q1, q2, q3 = _project(xyz, features, new_xyz, w1x, w1f)

    feats = []
    for (ps, qs, idx, k, kc, chunk, lp) in (
            (p1, q1, idxs[0], _KS[0], 16, 512, params[0]),
            (p2, q2, idxs[1], _KS[1], 8, 512, params[1]),
            (p3, q3, idxs[2], _KS[2], 8, 512, params[2])):
        rows_total = _B * k * _S
        table = ps.reshape(_B * _N, 128)
        g = _sc_gather(table, idx, rows_total, 128, chunk)
        feats.append(_mlp_scale(g, qs, lp, k, kc))

    new_feat = jnp.concatenate(feats, axis=-1)
    return new_xyz, new_feat


# speedup vs baseline: 1.6514x; 1.6514x over previous
import functools

import jax
import jax.numpy as jnp
from jax import lax
from jax.experimental import pallas as pl
from jax.experimental.pallas import tpu as pltpu
from jax.experimental.pallas import tpu_sc as plsc

_B = 8
_N = 2048
_S = 512
_RADII = (0.2, 0.4, 0.8)
_KS = (16, 32, 64)
_EPS = 1e-5


def _fps_body(xt_ref, nxo_ref):
    x = xt_ref[0]
    y = xt_ref[1]
    z = xt_ref[2]
    iota = lax.broadcasted_iota(jnp.int32, (_B, _N), 1)

    def step(i, carry):
        dist, far = carry
        oh = (iota == far).astype(jnp.float32)
        cx = jnp.sum(x * oh, axis=1, keepdims=True)
        cy = jnp.sum(y * oh, axis=1, keepdims=True)
        cz = jnp.sum(z * oh, axis=1, keepdims=True)
        nxo_ref[pl.ds(i, 1)] = jnp.concatenate([cx, cy, cz], axis=1)[None]
        dx = x - cx
        dy = y - cy
        dz = z - cz
        d = dx * dx + dy * dy + dz * dz
        dist = jnp.minimum(dist, d)
        m = jnp.max(dist, axis=1, keepdims=True)
        far = jnp.min(jnp.where(dist == m, iota, _N), axis=1, keepdims=True)
        return dist, far

    dist0 = jnp.full((_B, _N), jnp.inf, dtype=jnp.float32)
    far0 = jnp.zeros((_B, 1), dtype=jnp.int32)
    lax.fori_loop(0, _S, step, (dist0, far0))


def _fps(xyzt):
    return pl.pallas_call(
        _fps_body,
        out_shape=jax.ShapeDtypeStruct((_S, _B, 3), jnp.float32),
    )(xyzt)


def _bq_body(xt_ref, qt_ref, o1_ref, o2_ref, o3_ref):
    b = pl.program_id(0)
    xt = xt_ref[0]
    qt = qt_ref[0]
    xx = jnp.sum(xt * xt, axis=0)
    qq = jnp.sum(qt * qt, axis=0)
    prod = lax.dot_general(xt, qt, (((0,), (0,)), ((), ())),
                           preferred_element_type=jnp.float32)
    d = -2.0 * prod + qq[None, :] + xx[:, None]
    sq = jnp.maximum(d, 0.0)

    li = lax.broadcasted_iota(jnp.int32, (128, 128), 0)
    lj = lax.broadcasted_iota(jnp.int32, (128, 128), 1)
    lt128b = jnp.broadcast_to(((li <= lj).astype(jnp.float32))[None],
                              (16, 128, 128))
    si = lax.broadcasted_iota(jnp.int32, (16, 16), 0)
    sj = lax.broadcasted_iota(jnp.int32, (16, 16), 1)
    lts16 = (si < sj).astype(jnp.float32)

    for o_ref, r, k in ((o1_ref, _RADII[0], _KS[0]),
                        (o2_ref, _RADII[1], _KS[1]),
                        (o3_ref, _RADII[2], _KS[2])):
        mask = (sq <= r * r).astype(jnp.float32)
        m3 = mask.reshape(16, 128, _S)
        within = lax.dot_general(lt128b, m3, (((1,), (1,)), ((0,), (0,))),
                                 preferred_element_type=jnp.float32)
        tot = within[:, 127, :]
        carry = lax.dot_general(lts16, tot, (((0,), (0,)), ((), ())),
                                preferred_element_type=jnp.float32)
        c = (within + carry[:, None, :]).reshape(_N, _S)
        total = jnp.sum(tot, axis=0)

        def jstep(j, _, o_ref=o_ref, c=c):
            cnt = jnp.sum((c <= j.astype(jnp.float32)).astype(jnp.float32),
                          axis=0)
            o_ref[0, pl.ds(j, 1)] = cnt.astype(jnp.int32).reshape(1, _S)
            return 0

        lax.fori_loop(0, k, jstep, 0)
        nj = o_ref[0]
        jrow = lax.broadcasted_iota(jnp.int32, (k, _S), 0)
        trow = total.astype(jnp.int32).reshape(1, _S)
        o_ref[0] = jnp.where(jrow < trow, nj, nj[0:1]) + b * _N


def _ball_query(xt, qt):
    outs = [jax.ShapeDtypeStruct((_B, k, _S), jnp.int32) for k in _KS]
    return pl.pallas_call(
        _bq_body,
        grid=(_B,),
        in_specs=[
            pl.BlockSpec((1, 3, _N), lambda b: (b, 0, 0)),
            pl.BlockSpec((1, 3, _S), lambda b: (b, 0, 0)),
        ],
        out_specs=[pl.BlockSpec((1, k, _S), lambda b: (b, 0, 0)) for k in _KS],
        out_shape=outs,
    )(xt, qt)


_C1S = (32, 64, 64)
_SPLITS = (0, 32, 96, 160)


def _proj_body(xyz_ref, feat_ref, nx_ref, w1x_ref, w1f_ref,
               p1_ref, p2_ref, p3_ref, q1_ref, q2_ref, q3_ref):
    w1x = w1x_ref[...]
    w1f = w1f_ref[...]
    p = (lax.dot_general(xyz_ref[0], w1x, (((1,), (0,)), ((), ())),
                         preferred_element_type=jnp.float32)
         + lax.dot_general(feat_ref[0], w1f, (((1,), (0,)), ((), ())),
                           preferred_element_type=jnp.float32))
    q = lax.dot_general(nx_ref[0], w1x, (((1,), (0,)), ((), ())),
                        preferred_element_type=jnp.float32)
    for o_ref, lo, hi in ((p1_ref, 0, 32), (p2_ref, 32, 96), (p3_ref, 96, 160)):
        pad = jnp.zeros((_N, 128 - (hi - lo)), dtype=jnp.float32)
        o_ref[0] = jnp.concatenate([p[:, lo:hi], pad], axis=1)
    for o_ref, lo, hi in ((q1_ref, 0, 32), (q2_ref, 32, 96), (q3_ref, 96, 160)):
        o_ref[0] = q[:, lo:hi]


def _project(xyz, feat, nxyz, w1x, w1f):
    outs = ([jax.ShapeDtypeStruct((_B, _N, 128), jnp.float32) for c in _C1S]
            + [jax.ShapeDtypeStruct((_B, _S, c), jnp.float32) for c in _C1S])
    return pl.pallas_call(
        _proj_body,
        grid=(_B,),
        in_specs=[
            pl.BlockSpec((1, _N, 3), lambda b: (b, 0, 0)),
            pl.BlockSpec((1, _N, 64), lambda b: (b, 0, 0)),
            pl.BlockSpec((1, _S, 3), lambda b: (b, 0, 0)),
            pl.BlockSpec((3, 160), lambda b: (0, 0)),
            pl.BlockSpec((64, 160), lambda b: (0, 0)),
        ],
        out_specs=(
            [pl.BlockSpec((1, _N, 128), lambda b: (b, 0, 0)) for c in _C1S]
            + [pl.BlockSpec((1, _S, c), lambda b: (b, 0, 0)) for c in _C1S]),
        out_shape=outs,
    )(xyz, feat, nxyz, w1x, w1f)


def _sc_gather(table, idx, rows_total, c, chunk):
    nw = 32
    per_w = rows_total // nw
    n_chunks = per_w // chunk
    mesh = plsc.VectorSubcoreMesh(core_axis_name="c", subcore_axis_name="s")

    @functools.partial(
        pl.kernel, mesh=mesh,
        out_type=jax.ShapeDtypeStruct((rows_total, c), jnp.float32),
        scratch_types=[
            pltpu.VMEM((chunk,), jnp.int32),
            pltpu.VMEM((chunk, c), jnp.float32),
            pltpu.SemaphoreType.DMA,
        ],
    )
    def gk(table_hbm, idx_hbm, out_hbm, idx_v, rows_v, sem):
        wid = lax.axis_index("s") * 2 + lax.axis_index("c")
        base0 = wid * per_w

        def step(t, _):
            base = base0 + t * chunk
            pltpu.sync_copy(idx_hbm.at[pl.ds(base, chunk)], idx_v)
            pltpu.async_copy(table_hbm.at[idx_v], rows_v, sem).wait()
            pltpu.sync_copy(rows_v, out_hbm.at[pl.ds(base, chunk)])
            return 0

        lax.fori_loop(0, n_chunks, step, 0)

    return gk(table, idx)


def _bn_relu(y, st, gb, n_elems):
    mean = st[0:1] / n_elems
    var = st[1:2] / n_elems - mean * mean
    inv = lax.rsqrt(var + _EPS)
    return jnp.maximum((y - mean) * (inv * gb[0:1]) + gb[1:2], 0.0)


def _qtile(q_ref, kc):
    q = q_ref[0]
    return jnp.broadcast_to(q[None], (kc, _S, q.shape[-1])).reshape(
        kc * _S, q.shape[-1])


def _colstats(y):
    s = jnp.sum(y, axis=0, keepdims=True)
    s2 = jnp.sum(y * y, axis=0, keepdims=True)
    return jnp.concatenate([s, s2], axis=0)


def _acc_stats(o_ref, st):
    first = (pl.program_id(0) == 0) & (pl.program_id(1) == 0)

    @pl.when(first)
    def _():
        o_ref[...] = st

    @pl.when(jnp.logical_not(first))
    def _():
        o_ref[...] = o_ref[...] + st


def _forward(refs, kc, n_elems, depth):
    g_ref, q_ref = refs[0], refs[1]
    c1 = q_ref.shape[-1]
    h = g_ref[...][:, :c1] - _qtile(q_ref, kc)
    i = 2
    for layer in range(depth):
        st = refs[i][...]
        gb = refs[i + 1][...]
        h = _bn_relu(h, st, gb, n_elems)
        i += 2
        if layer < depth - 1:
            wt = refs[i][...]
            i += 1
            h = lax.dot_general(h, wt, (((1,), (0,)), ((), ())),
                                preferred_element_type=jnp.float32)
    return h


def _make_e1(kc):
    def body(g_ref, q_ref, o_ref):
        c1 = q_ref.shape[-1]
        y = g_ref[...][:, :c1] - _qtile(q_ref, kc)
        _acc_stats(o_ref, _colstats(y))
    return body


def _make_e2(kc, n_elems):
    def body(g_ref, q_ref, st1_ref, gb1_ref, w2t_ref, o_ref):
        h = _forward((g_ref, q_ref, st1_ref, gb1_ref), kc, n_elems, 1)
        y2 = lax.dot_general(h, w2t_ref[...], (((1,), (0,)), ((), ())),
                             preferred_element_type=jnp.float32)
        _acc_stats(o_ref, _colstats(y2))
    return body


def _make_e3(kc, n_elems):
    def body(g_ref, q_ref, st1_ref, gb1_ref, w2t_ref, st2_ref, gb2_ref,
             w3t_ref, o_ref):
        h = _forward((g_ref, q_ref, st1_ref, gb1_ref, w2t_ref, st2_ref,
                      gb2_ref), kc, n_elems, 2)
        y3 = lax.dot_general(h, w3t_ref[...], (((1,), (0,)), ((), ())),
                             preferred_element_type=jnp.float32)
        _acc_stats(o_ref, _colstats(y3))
    return body


def _make_e4(kc, n_elems):
    def body(g_ref, q_ref, st1_ref, gb1_ref, w2t_ref, st2_ref, gb2_ref,
             w3t_ref, st3_ref, gb3_ref, o_ref):
        h = _forward((g_ref, q_ref, st1_ref, gb1_ref, w2t_ref, st2_ref,
                      gb2_ref, w3t_ref, st3_ref, gb3_ref), kc, n_elems, 3)
        part = jnp.max(h.reshape(kc, _S, h.shape[-1]), axis=0)[None]

        @pl.when(pl.program_id(1) == 0)
        def _():
            o_ref[...] = part

        @pl.when(pl.program_id(1) != 0)
        def _():
            o_ref[...] = jnp.maximum(o_ref[...], part)
    return body


def _mlp_scale(g, q, layer_params, k, kc):
    n_elems = float(_B * k * _S)
    (w1, g1, b1), (w2, g2, b2), (w3, g3, b3) = layer_params
    c1, c2, c3 = w1.shape[0], w2.shape[0], w3.shape[0]
    gb1 = jnp.stack([g1, b1])
    gb2 = jnp.stack([g2, b2])
    gb3 = jnp.stack([g3, b3])
    w2t = jnp.transpose(w2)
    w3t = jnp.transpose(w3)
    grid = (_B, k // kc)
    rows = kc * _S

    g_spec = pl.BlockSpec((rows, 128),
                          lambda b, t, nt=k // kc: (b * nt + t, 0))
    q_spec = pl.BlockSpec((1, _S, c1), lambda b, t: (b, 0, 0))

    def full(shape):
        nd = len(shape)
        return pl.BlockSpec(shape, lambda b, t, nd=nd: (0,) * nd)

    st1 = pl.pallas_call(
        _make_e1(kc), grid=grid,
        in_specs=[g_spec, q_spec],
        out_specs=pl.BlockSpec((2, c1), lambda b, t: (0, 0)),
        out_shape=jax.ShapeDtypeStruct((2, c1), jnp.float32),
    )(g, q)

    st2 = pl.pallas_call(
        _make_e2(kc, n_elems), grid=grid,
        in_specs=[g_spec, q_spec, full((2, c1)), full((2, c1)),
                  full((c1, c2))],
        out_specs=pl.BlockSpec((2, c2), lambda b, t: (0, 0)),
        out_shape=jax.ShapeDtypeStruct((2, c2), jnp.float32),
    )(g, q, st1, gb1, w2t)

    st3 = pl.pallas_call(
        _make_e3(kc, n_elems), grid=grid,
        in_specs=[g_spec, q_spec, full((2, c1)), full((2, c1)),
                  full((c1, c2)), full((2, c2)), full((2, c2)),
                  full((c2, c3))],
        out_specs=pl.BlockSpec((2, c3), lambda b, t: (0, 0)),
        out_shape=jax.ShapeDtypeStruct((2, c3), jnp.float32),
    )(g, q, st1, gb1, w2t, st2, gb2, w3t)

    out = pl.pallas_call(
        _make_e4(kc, n_elems), grid=grid,
        in_specs=[g_spec, q_spec, full((2, c1)), full((2, c1)),
                  full((c1, c2)), full((2, c2)), full((2, c2)),
                  full((c2, c3)), full((2, c3)), full((2, c3))],
        out_specs=pl.BlockSpec((1, _S, c3), lambda b, t: (b, 0, 0)),
        out_shape=jax.ShapeDtypeStruct((_B, _S, c3), jnp.float32),
    )(g, q, st1, gb1, w2t, st2, gb2, w3t, st3, gb3)
    return out


def kernel(xyz, features, params):
    xyzt = jnp.transpose(xyz, (2, 0, 1))
    nx_sb3 = _fps(xyzt)
    new_xyz = jnp.transpose(nx_sb3, (1, 0, 2))

    xt = jnp.transpose(xyz, (0, 2, 1))
    qt = jnp.transpose(new_xyz, (0, 2, 1))
    idxs_ks = _ball_query(xt, qt)
    idxs = [ix.reshape(-1) for ix in idxs_ks]

    w1cat = jnp.concatenate([lp[0][0] for lp in params], axis=0)
    w1x = jnp.transpose(w1cat[:, :3])
    w1f = jnp.transpose(w1cat[:, 3:])
    p1, p2, p3, q1, q2, q3 = _project(xyz, features, new_xyz, w1x, w1f)

    feats = []
    for (ps, qs, idx, k, kc, chunk, lp) in (
            (p1, q1, idxs[0], _KS[0], 16, 512, params[0]),
            (p2, q2, idxs[1], _KS[1], 8, 512, params[1]),
            (p3, q3, idxs[2], _KS[2], 8, 512, params[2])):
        rows_total = _B * k * _S
        table = ps.reshape(_B * _N, 128)
        g = _sc_gather(table, idx, rows_total, 128, chunk)
        feats.append(_mlp_scale(g, qs, lp, k, kc))

    new_feat = jnp.concatenate(feats, axis=-1)
    return new_xyz, new_feat
